# baseline (speedup 1.0000x reference)
import functools

import jax
import jax.numpy as jnp
from jax.experimental import pallas as pl
from jax.experimental.pallas import tpu as pltpu

N_EMBED = 1024
HIDDEN = 4 * N_EMBED
NUM_EXPERTS = 8
TOP_K = 2
SEQ = 2048
ROWS = 512
NEG = -1e30


def _moe_body(x_ref, w1t_ref, b1_ref, w2t_ref, b2_ref, wgt_ref, bg_ref,
              out_ref, wall_ref):
    e = pl.program_id(1)

    @pl.when(e == 0)
    def _gate():
        xf = x_ref[...]
        logits = jnp.dot(xf, wgt_ref[...],
                         preferred_element_type=jnp.float32) + bg_ref[...]
        lanes = jax.lax.broadcasted_iota(jnp.int32, (ROWS, 128), 1)
        m1 = jnp.max(logits, axis=1, keepdims=True)
        i1 = jnp.min(jnp.where(logits == m1, lanes, 128), axis=1, keepdims=True)
        l2 = jnp.where(lanes == i1, NEG, logits)
        m2 = jnp.max(l2, axis=1, keepdims=True)
        i2 = jnp.min(jnp.where(l2 == m2, lanes, 128), axis=1, keepdims=True)
        s = jnp.sum(jnp.exp(logits - m1), axis=1, keepdims=True)
        wtop1 = 1.0 / s
        wtop2 = jnp.exp(m2 - m1) / s
        wall_ref[...] = (jnp.where(lanes == i1, wtop1, 0.0)
                         + jnp.where(lanes == i2, wtop2, 0.0))
        out_ref[...] = jnp.zeros_like(out_ref)

    xbf = x_ref[...].astype(jnp.bfloat16)
    h = jnp.dot(xbf, w1t_ref[0], preferred_element_type=jnp.float32)
    h = jax.nn.relu(h + b1_ref[0])
    y = jnp.dot(h.astype(jnp.bfloat16), w2t_ref[0],
                preferred_element_type=jnp.float32) + b2_ref[0]
    lanes = jax.lax.broadcasted_iota(jnp.int32, (ROWS, 128), 1)
    we = jnp.sum(jnp.where(lanes == e, wall_ref[...], 0.0), axis=1,
                 keepdims=True)
    out_ref[...] += we * y


def kernel(x, W1, b1, W2, b2, Wg, bg):
    xs = x.reshape(SEQ, N_EMBED)
    w1t = jnp.swapaxes(W1, 1, 2).astype(jnp.bfloat16)
    w2t = jnp.swapaxes(W2, 1, 2).astype(jnp.bfloat16)
    wgt = jnp.pad(Wg.T, ((0, 0), (0, 128 - NUM_EXPERTS)))
    bgp = jnp.pad(bg, (0, 128 - NUM_EXPERTS),
                  constant_values=NEG).reshape(1, 128)

    nr = SEQ // ROWS
    grid = (nr, NUM_EXPERTS)
    out = pl.pallas_call(
        _moe_body,
        grid=grid,
        in_specs=[
            pl.BlockSpec((ROWS, N_EMBED), lambda r, e: (r, 0)),
            pl.BlockSpec((1, N_EMBED, HIDDEN), lambda r, e: (e, 0, 0)),
            pl.BlockSpec((1, 1, HIDDEN), lambda r, e: (e, 0, 0)),
            pl.BlockSpec((1, HIDDEN, N_EMBED), lambda r, e: (e, 0, 0)),
            pl.BlockSpec((1, 1, N_EMBED), lambda r, e: (e, 0, 0)),
            pl.BlockSpec((N_EMBED, 128), lambda r, e: (0, 0)),
            pl.BlockSpec((1, 128), lambda r, e: (0, 0)),
        ],
        out_specs=pl.BlockSpec((ROWS, N_EMBED), lambda r, e: (r, 0)),
        out_shape=jax.ShapeDtypeStruct((SEQ, N_EMBED), jnp.float32),
        scratch_shapes=[pltpu.VMEM((ROWS, 128), jnp.float32)],
    )(xs, w1t, b1.reshape(NUM_EXPERTS, 1, HIDDEN), w2t,
      b2.reshape(NUM_EXPERTS, 1, N_EMBED), wgt, bgp)
    return out.reshape(x.shape)
